# baseline (speedup 1.0000x reference)
import functools

import jax
import jax.numpy as jnp
from jax import lax
from jax.experimental import pallas as pl
from jax.experimental.pallas import tpu as pltpu
from jax.experimental.pallas import tpu_sc as plsc

E = 8
TOP_K = 2
D = 1024
F = 512
FS = 1024
ROUTED_SCALE = 2.5

T = 2048
A = T * TOP_K
BLK = 256
NBLK = A // BLK + E
NSLOT = NBLK * BLK
TBLK = 256
NT = T // TBLK


def _inc_cumsum0(m):
    c = m
    d = 1
    while d < c.shape[0]:
        z = jnp.zeros((d, c.shape[1]), c.dtype)
        c = c + jnp.concatenate([z, c[:-d]], axis=0)
        d *= 2
    return c


def _col_to_2d(col, rows, lanes):
    n = col.shape[0]
    lane = lax.broadcasted_iota(jnp.int32, (n, lanes), 1)
    sub = lax.broadcasted_iota(jnp.int32, (n, lanes), 0)
    b = jnp.where(lane == sub % lanes, col, 0.0)
    return jnp.sum(b.reshape(rows, lanes, lanes), axis=1)


def _rows_to_col(w2):
    r = w2.shape[0]
    n = r * 128
    rep = jnp.broadcast_to(w2[:, None, :], (r, 128, 128)).reshape(n, 128)
    lane = lax.broadcasted_iota(jnp.int32, (n, 128), 1)
    sub = lax.broadcasted_iota(jnp.int32, (n, 128), 0)
    return jnp.sum(jnp.where(lane == sub % 128, rep, 0.0), axis=-1,
                   keepdims=True)


def _plan_body(x_ref, wr_ref, inv0_ref, inv1_ref, wt0_ref, wt1_ref,
               beid_ref):
    x = x_ref[...]
    logits = jnp.dot(x, wr_ref[...], preferred_element_type=jnp.float32)
    s = jax.nn.softmax(logits, axis=-1)
    lane8 = lax.broadcasted_iota(jnp.int32, s.shape, 1)
    m1 = jnp.max(s, axis=-1, keepdims=True)
    i1 = jnp.min(jnp.where(s == m1, lane8, E), axis=-1, keepdims=True)
    oh1 = (lane8 == i1).astype(jnp.float32)
    s2 = jnp.where(lane8 == i1, -jnp.inf, s)
    m2 = jnp.max(s2, axis=-1, keepdims=True)
    i2 = jnp.min(jnp.where(s2 == m2, lane8, E), axis=-1, keepdims=True)
    oh2 = (lane8 == i2).astype(jnp.float32)
    v1 = jnp.sum(s * oh1, axis=-1, keepdims=True)
    v2 = jnp.sum(s * oh2, axis=-1, keepdims=True)

    inc1 = _inc_cumsum0(oh1)
    inc2 = _inc_cumsum0(oh2)
    exc1 = inc1 - oh1
    exc2 = inc2 - oh2
    cnt1 = inc1[T - 1:T, :]
    cnt2 = inc2[T - 1:T, :]
    cnt = cnt1 + cnt2
    nb = jnp.ceil(cnt * (1.0 / BLK))
    ii = lax.broadcasted_iota(jnp.int32, (E, E), 0)
    jj = lax.broadcasted_iota(jnp.int32, (E, E), 1)
    tri = (ii <= jj).astype(jnp.float32)
    bcum = jnp.dot(nb, tri)
    boff = bcum - nb
    soff = boff * float(BLK)

    soff1 = jnp.sum(oh1 * soff, axis=-1, keepdims=True)
    soff2 = jnp.sum(oh2 * (soff + cnt1), axis=-1, keepdims=True)
    rank1 = jnp.sum(oh1 * exc1, axis=-1, keepdims=True)
    rank2 = jnp.sum(oh2 * exc2, axis=-1, keepdims=True)
    slot0 = soff1 + rank1
    slot1 = soff2 + rank2

    inv0_ref[...] = _col_to_2d(slot0, T // 128, 128).astype(jnp.int32)
    inv1_ref[...] = _col_to_2d(slot1, T // 128, 128).astype(jnp.int32)
    wt0_ref[...] = _col_to_2d(v1, T // 128, 128)
    wt1_ref[...] = _col_to_2d(v2, T // 128, 128)

    biota = lax.broadcasted_iota(jnp.int32, (1, 128), 1).astype(jnp.float32)
    acc = jnp.zeros((1, 128), jnp.float32)
    for e in range(E):
        bce = jnp.sum(jnp.where(lane8[:1] == e, bcum, 0.0))
        acc = acc + (biota >= bce).astype(jnp.float32)
    beid_ref[...] = jnp.minimum(acc, float(E - 1)).astype(jnp.int32)


def _gemm_body(beid_ref, sg_ref, su_ref, sd_ref, xs_ref, wg_ref, wu_ref,
               wd_ref, ys_ref):
    b = pl.program_id(0)
    e = beid_ref[b]
    xb = xs_ref[...]
    g = jnp.dot(xb, wg_ref[0], preferred_element_type=jnp.float32) * sg_ref[e]
    u = jnp.dot(xb, wu_ref[0], preferred_element_type=jnp.float32) * su_ref[e]
    h = (jax.nn.silu(g) * u).astype(jnp.bfloat16)
    ys_ref[...] = (jnp.dot(h, wd_ref[0], preferred_element_type=jnp.float32)
                   * sd_ref[e]).astype(jnp.bfloat16)


def _comb_body(x_ref, wg_ref, wu_ref, wd_ref, y0_ref, y1_ref, wt0_ref,
               wt1_ref, out_ref):
    x = x_ref[...].astype(jnp.bfloat16)
    g = jnp.dot(x, wg_ref[...], preferred_element_type=jnp.float32)
    u = jnp.dot(x, wu_ref[...], preferred_element_type=jnp.float32)
    h = (jax.nn.silu(g) * u).astype(jnp.bfloat16)
    sh = jnp.dot(h, wd_ref[...], preferred_element_type=jnp.float32)
    w0 = _rows_to_col(wt0_ref[0])
    w1 = _rows_to_col(wt1_ref[0])
    y0 = y0_ref[...].astype(jnp.float32)
    y1 = y1_ref[...].astype(jnp.float32)
    out_ref[...] = sh + (w0 * y0 + w1 * y1) * ROUTED_SCALE


def _sc_info():
    info = plsc.get_sparse_core_info()
    return info.num_cores, info.num_subcores


def _make_sc_scatter():
    nc, ns = _sc_info()
    nw = nc * ns
    tpw = T // nw
    mesh = plsc.VectorSubcoreMesh(core_axis_name="c", subcore_axis_name="s")

    @functools.partial(
        pl.kernel, mesh=mesh,
        out_type=jax.ShapeDtypeStruct((NSLOT, D // 2), jnp.int32),
        scratch_types=[
            pltpu.VMEM((tpw,), jnp.int32),
            pltpu.VMEM((tpw,), jnp.int32),
            pltpu.VMEM((tpw, D // 2), jnp.int32),
            pltpu.SemaphoreType.DMA,
            pltpu.SemaphoreType.DMA,
        ],
    )
    def sc_scatter(x_hbm, inv0_hbm, inv1_hbm, xs_hbm, idx0_v, idx1_v,
                   rows_v, sem, sem2):
        wid = lax.axis_index("s") * nc + lax.axis_index("c")
        base = wid * tpw
        ld = pltpu.async_copy(x_hbm.at[pl.ds(base, tpw)], rows_v, sem2)
        pltpu.sync_copy(inv0_hbm.at[pl.ds(base, tpw)], idx0_v)
        pltpu.sync_copy(inv1_hbm.at[pl.ds(base, tpw)], idx1_v)
        ld.wait()
        c0 = pltpu.async_copy(rows_v, xs_hbm.at[idx0_v], sem)
        c1 = pltpu.async_copy(rows_v, xs_hbm.at[idx1_v], sem)
        c0.wait()
        c1.wait()

    return sc_scatter


def _make_sc_gather():
    nc, ns = _sc_info()
    nw = nc * ns
    apw = T // nw
    mesh = plsc.VectorSubcoreMesh(core_axis_name="c", subcore_axis_name="s")

    @functools.partial(
        pl.kernel, mesh=mesh,
        out_type=jax.ShapeDtypeStruct((A, D // 2), jnp.int32),
        scratch_types=[
            pltpu.VMEM((apw,), jnp.int32),
            pltpu.VMEM((apw,), jnp.int32),
            pltpu.VMEM((apw, D // 2), jnp.int32),
            pltpu.VMEM((apw, D // 2), jnp.int32),
            pltpu.SemaphoreType.DMA,
            pltpu.SemaphoreType.DMA,
        ],
    )
    def sc_gather(ys_hbm, inv0_hbm, inv1_hbm, ytm_hbm, idx0_v, idx1_v,
                  rows0_v, rows1_v, sem0, sem1):
        wid = lax.axis_index("s") * nc + lax.axis_index("c")
        base = wid * apw
        pltpu.sync_copy(inv0_hbm.at[pl.ds(base, apw)], idx0_v)
        g0 = pltpu.async_copy(ys_hbm.at[idx0_v], rows0_v, sem0)
        pltpu.sync_copy(inv1_hbm.at[pl.ds(base, apw)], idx1_v)
        g1 = pltpu.async_copy(ys_hbm.at[idx1_v], rows1_v, sem1)
        g0.wait()
        o0 = pltpu.async_copy(rows0_v, ytm_hbm.at[pl.ds(base, apw)], sem0)
        g1.wait()
        o1 = pltpu.async_copy(rows1_v, ytm_hbm.at[pl.ds(T + base, apw)], sem1)
        o0.wait()
        o1.wait()

    return sc_gather


def kernel(hidden_states, w_router, w_gate_fp8, w_up_fp8, w_down_fp8,
           s_g, s_u, s_d, w_sh_gate, w_sh_up, w_sh_down):
    shape = hidden_states.shape
    x = hidden_states.reshape(-1, shape[-1])

    inv0_2d, inv1_2d, wt0_2d, wt1_2d, beid_2d = pl.pallas_call(
        _plan_body,
        in_specs=[
            pl.BlockSpec((T, D), lambda: (0, 0)),
            pl.BlockSpec((D, E), lambda: (0, 0)),
        ],
        out_specs=[
            pl.BlockSpec((T // 128, 128), lambda: (0, 0)),
            pl.BlockSpec((T // 128, 128), lambda: (0, 0)),
            pl.BlockSpec((T // 128, 128), lambda: (0, 0)),
            pl.BlockSpec((T // 128, 128), lambda: (0, 0)),
            pl.BlockSpec((1, 128), lambda: (0, 0)),
        ],
        out_shape=[
            jax.ShapeDtypeStruct((T // 128, 128), jnp.int32),
            jax.ShapeDtypeStruct((T // 128, 128), jnp.int32),
            jax.ShapeDtypeStruct((T // 128, 128), jnp.float32),
            jax.ShapeDtypeStruct((T // 128, 128), jnp.float32),
            jax.ShapeDtypeStruct((1, 128), jnp.int32),
        ],
    )(x, w_router)

    inv0 = inv0_2d.reshape(T)
    inv1 = inv1_2d.reshape(T)
    beid = beid_2d[0, :NBLK]

    x_i32 = lax.bitcast_convert_type(
        x.astype(jnp.bfloat16).reshape(T, D // 2, 2), jnp.int32)
    xs_i32 = _make_sc_scatter()(x_i32, inv0, inv1)
    xs = lax.bitcast_convert_type(xs_i32, jnp.bfloat16).reshape(NSLOT, D)

    wg_bf = w_gate_fp8.astype(jnp.bfloat16)
    wu_bf = w_up_fp8.astype(jnp.bfloat16)
    wd_bf = w_down_fp8.astype(jnp.bfloat16)

    ys = pl.pallas_call(
        _gemm_body,
        grid_spec=pltpu.PrefetchScalarGridSpec(
            num_scalar_prefetch=4,
            grid=(NBLK,),
            in_specs=[
                pl.BlockSpec((BLK, D), lambda b, beid, sg, su, sd: (b, 0)),
                pl.BlockSpec((1, D, F),
                             lambda b, beid, sg, su, sd: (beid[b], 0, 0)),
                pl.BlockSpec((1, D, F),
                             lambda b, beid, sg, su, sd: (beid[b], 0, 0)),
                pl.BlockSpec((1, F, D),
                             lambda b, beid, sg, su, sd: (beid[b], 0, 0)),
            ],
            out_specs=pl.BlockSpec((BLK, D),
                                   lambda b, beid, sg, su, sd: (b, 0)),
        ),
        out_shape=jax.ShapeDtypeStruct((NSLOT, D), jnp.bfloat16),
    )(beid, s_g, s_u, s_d, xs, wg_bf, wu_bf, wd_bf)

    ys_i32 = lax.bitcast_convert_type(ys.reshape(NSLOT, D // 2, 2), jnp.int32)
    ytm_i32 = _make_sc_gather()(ys_i32, inv0, inv1)
    ytm = lax.bitcast_convert_type(ytm_i32, jnp.bfloat16).reshape(A, D)

    wsg_bf = w_sh_gate.astype(jnp.bfloat16)
    wsu_bf = w_sh_up.astype(jnp.bfloat16)
    wsd_bf = w_sh_down.astype(jnp.bfloat16)

    out = pl.pallas_call(
        _comb_body,
        grid=(NT,),
        in_specs=[
            pl.BlockSpec((TBLK, D), lambda t: (t, 0)),
            pl.BlockSpec((D, FS), lambda t: (0, 0)),
            pl.BlockSpec((D, FS), lambda t: (0, 0)),
            pl.BlockSpec((FS, D), lambda t: (0, 0)),
            pl.BlockSpec((TBLK, D), lambda t: (t, 0)),
            pl.BlockSpec((TBLK, D), lambda t: (t + NT, 0)),
            pl.BlockSpec((1, TBLK // 128, 128), lambda t: (t, 0, 0)),
            pl.BlockSpec((1, TBLK // 128, 128), lambda t: (t, 0, 0)),
        ],
        out_specs=pl.BlockSpec((TBLK, D), lambda t: (t, 0)),
        out_shape=jax.ShapeDtypeStruct((T, D), jnp.float32),
    )(x, wsg_bf, wsu_bf, wsd_bf, ytm, ytm,
      wt0_2d.reshape(NT, TBLK // 128, 128),
      wt1_2d.reshape(NT, TBLK // 128, 128))

    return out.reshape(shape)


# speedup vs baseline: 3.5776x; 3.5776x over previous
import functools

import jax
import jax.numpy as jnp
from jax import lax
from jax.experimental import pallas as pl
from jax.experimental.pallas import tpu as pltpu
from jax.experimental.pallas import tpu_sc as plsc

E = 8
TOP_K = 2
D = 1024
F = 512
FS = 1024
ROUTED_SCALE = 2.5

T = 2048
A = T * TOP_K
BLK = 256
NBLK = A // BLK + E
NSLOT = NBLK * BLK
TBLK = 256
NT = T // TBLK


def _inc_cumsum0(m):
    c = m
    d = 1
    while d < c.shape[0]:
        z = jnp.zeros((d, c.shape[1]), c.dtype)
        c = c + jnp.concatenate([z, c[:-d]], axis=0)
        d *= 2
    return c


def _col_to_2d(col, rows, lanes):
    n = col.shape[0]
    lane = lax.broadcasted_iota(jnp.int32, (n, lanes), 1)
    sub = lax.broadcasted_iota(jnp.int32, (n, lanes), 0)
    b = jnp.where(lane == sub % lanes, col, 0.0)
    return jnp.sum(b.reshape(rows, lanes, lanes), axis=1)


def _rows_to_col(w2):
    r = w2.shape[0]
    n = r * 128
    rep = jnp.broadcast_to(w2[:, None, :], (r, 128, 128)).reshape(n, 128)
    lane = lax.broadcasted_iota(jnp.int32, (n, 128), 1)
    sub = lax.broadcasted_iota(jnp.int32, (n, 128), 0)
    return jnp.sum(jnp.where(lane == sub % 128, rep, 0.0), axis=-1,
                   keepdims=True)


def _plan_body(x_ref, wr_ref, inv0_ref, inv1_ref, wt0_ref, wt1_ref,
               beid_ref):
    x = x_ref[...]
    logits = jnp.dot(x, wr_ref[...], preferred_element_type=jnp.float32)
    s = jax.nn.softmax(logits, axis=-1)
    lane8 = lax.broadcasted_iota(jnp.int32, s.shape, 1)
    m1 = jnp.max(s, axis=-1, keepdims=True)
    i1 = jnp.min(jnp.where(s == m1, lane8, E), axis=-1, keepdims=True)
    oh1 = (lane8 == i1).astype(jnp.float32)
    s2 = jnp.where(lane8 == i1, -jnp.inf, s)
    m2 = jnp.max(s2, axis=-1, keepdims=True)
    i2 = jnp.min(jnp.where(s2 == m2, lane8, E), axis=-1, keepdims=True)
    oh2 = (lane8 == i2).astype(jnp.float32)
    v1 = jnp.sum(s * oh1, axis=-1, keepdims=True)
    v2 = jnp.sum(s * oh2, axis=-1, keepdims=True)

    inc1 = _inc_cumsum0(oh1)
    inc2 = _inc_cumsum0(oh2)
    exc1 = inc1 - oh1
    exc2 = inc2 - oh2
    cnt1 = inc1[T - 1:T, :]
    cnt2 = inc2[T - 1:T, :]
    cnt = cnt1 + cnt2
    nb = jnp.ceil(cnt * (1.0 / BLK))
    ii = lax.broadcasted_iota(jnp.int32, (E, E), 0)
    jj = lax.broadcasted_iota(jnp.int32, (E, E), 1)
    tri = (ii <= jj).astype(jnp.float32)
    bcum = jnp.dot(nb, tri)
    boff = bcum - nb
    soff = boff * float(BLK)

    soff1 = jnp.sum(oh1 * soff, axis=-1, keepdims=True)
    soff2 = jnp.sum(oh2 * (soff + cnt1), axis=-1, keepdims=True)
    rank1 = jnp.sum(oh1 * exc1, axis=-1, keepdims=True)
    rank2 = jnp.sum(oh2 * exc2, axis=-1, keepdims=True)
    slot0 = soff1 + rank1
    slot1 = soff2 + rank2

    inv0_ref[...] = _col_to_2d(slot0, T // 128, 128).astype(jnp.int32)
    inv1_ref[...] = _col_to_2d(slot1, T // 128, 128).astype(jnp.int32)
    wt0_ref[...] = v1 * ROUTED_SCALE
    wt1_ref[...] = v2 * ROUTED_SCALE

    biota = lax.broadcasted_iota(jnp.int32, (1, 128), 1).astype(jnp.float32)
    acc = jnp.zeros((1, 128), jnp.float32)
    for e in range(E):
        bce = jnp.sum(jnp.where(lane8[:1] == e, bcum, 0.0))
        acc = acc + (biota >= bce).astype(jnp.float32)
    beid_ref[...] = jnp.minimum(acc, float(E - 1)).astype(jnp.int32)


def _gemm_body(beid_ref, sg_ref, su_ref, sd_ref, xs_ref, wg_ref, wu_ref,
               wd_ref, ys_ref):
    b = pl.program_id(0)
    e = beid_ref[b]
    xb = xs_ref[...].astype(jnp.bfloat16)
    g = jnp.dot(xb, wg_ref[0], preferred_element_type=jnp.float32) * sg_ref[e]
    u = jnp.dot(xb, wu_ref[0], preferred_element_type=jnp.float32) * su_ref[e]
    h = (jax.nn.silu(g) * u).astype(jnp.bfloat16)
    ys_ref[...] = (jnp.dot(h, wd_ref[0], preferred_element_type=jnp.float32)
                   * sd_ref[e])


def _shared_body(x_ref, wg_ref, wu_ref, wd_ref, sh_ref):
    x = x_ref[...].astype(jnp.bfloat16)
    g = jnp.dot(x, wg_ref[...], preferred_element_type=jnp.float32)
    u = jnp.dot(x, wu_ref[...], preferred_element_type=jnp.float32)
    h = (jax.nn.silu(g) * u).astype(jnp.bfloat16)
    sh_ref[...] = jnp.dot(h, wd_ref[...],
                          preferred_element_type=jnp.float32).astype(
                              jnp.bfloat16)


def _comb_body(sh_ref, y0_ref, y1_ref, wt0_ref, wt1_ref, out_ref):
    w0 = wt0_ref[...]
    w1 = wt1_ref[...]
    y0 = y0_ref[...]
    y1 = y1_ref[...]
    out_ref[...] = sh_ref[...].astype(jnp.float32) + w0 * y0 + w1 * y1


def _sc_info():
    info = plsc.get_sparse_core_info()
    return info.num_cores, info.num_subcores


def _make_sc_scatter():
    nc, ns = _sc_info()
    nw = nc * ns
    tpw = T // nw
    mesh = plsc.VectorSubcoreMesh(core_axis_name="c", subcore_axis_name="s")

    @functools.partial(
        pl.kernel, mesh=mesh,
        out_type=jax.ShapeDtypeStruct((NSLOT, D), jnp.float32),
        scratch_types=[
            pltpu.VMEM((tpw,), jnp.int32),
            pltpu.VMEM((tpw,), jnp.int32),
            pltpu.VMEM((tpw, D), jnp.float32),
            pltpu.SemaphoreType.DMA,
            pltpu.SemaphoreType.DMA,
        ],
    )
    def sc_scatter(x_hbm, inv0_hbm, inv1_hbm, xs_hbm, idx0_v, idx1_v,
                   rows_v, sem, sem2):
        wid = lax.axis_index("s") * nc + lax.axis_index("c")
        base = wid * tpw
        ld = pltpu.async_copy(x_hbm.at[pl.ds(base, tpw)], rows_v, sem2)
        pltpu.sync_copy(inv0_hbm.at[pl.ds(base, tpw)], idx0_v)
        pltpu.sync_copy(inv1_hbm.at[pl.ds(base, tpw)], idx1_v)
        ld.wait()
        c0 = pltpu.async_copy(rows_v, xs_hbm.at[idx0_v], sem)
        c1 = pltpu.async_copy(rows_v, xs_hbm.at[idx1_v], sem)
        c0.wait()
        c1.wait()

    return sc_scatter


def _make_sc_gather():
    nc, ns = _sc_info()
    nw = nc * ns
    apw = T // nw
    mesh = plsc.VectorSubcoreMesh(core_axis_name="c", subcore_axis_name="s")

    ch = apw // 2

    @functools.partial(
        pl.kernel, mesh=mesh,
        out_type=jax.ShapeDtypeStruct((A, D), jnp.float32),
        scratch_types=[
            pltpu.VMEM((ch,), jnp.int32),
            pltpu.VMEM((ch,), jnp.int32),
            pltpu.VMEM((ch,), jnp.int32),
            pltpu.VMEM((ch,), jnp.int32),
            pltpu.VMEM((ch, D), jnp.float32),
            pltpu.VMEM((ch, D), jnp.float32),
            pltpu.SemaphoreType.DMA,
            pltpu.SemaphoreType.DMA,
            pltpu.SemaphoreType.DMA,
        ],
    )
    def sc_gather(ys_hbm, inv0_hbm, inv1_hbm, ytm_hbm, ia_v, ib_v, ic_v,
                  id_v, rows0_v, rows1_v, sg, so0, so1):
        wid = lax.axis_index("s") * nc + lax.axis_index("c")
        base = wid * apw
        pltpu.sync_copy(inv0_hbm.at[pl.ds(base, ch)], ia_v)
        pltpu.sync_copy(inv0_hbm.at[pl.ds(base + ch, ch)], ib_v)
        pltpu.sync_copy(inv1_hbm.at[pl.ds(base, ch)], ic_v)
        pltpu.sync_copy(inv1_hbm.at[pl.ds(base + ch, ch)], id_v)
        bufs = (rows0_v, rows1_v)
        osems = (so0, so1)
        descs = ((ia_v, base), (ib_v, base + ch),
                 (ic_v, T + base), (id_v, T + base + ch))
        outs = [None, None]
        for i, (iv, dst) in enumerate(descs):
            b = i % 2
            if outs[b] is not None:
                outs[b].wait()
            pltpu.async_copy(ys_hbm.at[iv], bufs[b], sg).wait()
            outs[b] = pltpu.async_copy(
                bufs[b], ytm_hbm.at[pl.ds(dst, ch)], osems[b])
        outs[0].wait()
        outs[1].wait()

    return sc_gather


def kernel(hidden_states, w_router, w_gate_fp8, w_up_fp8, w_down_fp8,
           s_g, s_u, s_d, w_sh_gate, w_sh_up, w_sh_down):
    shape = hidden_states.shape
    x = hidden_states.reshape(-1, shape[-1])

    inv0_2d, inv1_2d, wt0_col, wt1_col, beid_2d = pl.pallas_call(
        _plan_body,
        in_specs=[
            pl.BlockSpec((T, D), lambda: (0, 0)),
            pl.BlockSpec((D, E), lambda: (0, 0)),
        ],
        out_specs=[
            pl.BlockSpec((T // 128, 128), lambda: (0, 0)),
            pl.BlockSpec((T // 128, 128), lambda: (0, 0)),
            pl.BlockSpec((T, 1), lambda: (0, 0)),
            pl.BlockSpec((T, 1), lambda: (0, 0)),
            pl.BlockSpec((1, 128), lambda: (0, 0)),
        ],
        out_shape=[
            jax.ShapeDtypeStruct((T // 128, 128), jnp.int32),
            jax.ShapeDtypeStruct((T // 128, 128), jnp.int32),
            jax.ShapeDtypeStruct((T, 1), jnp.float32),
            jax.ShapeDtypeStruct((T, 1), jnp.float32),
            jax.ShapeDtypeStruct((1, 128), jnp.int32),
        ],
    )(x, w_router)

    inv0 = inv0_2d.reshape(T)
    inv1 = inv1_2d.reshape(T)
    beid = beid_2d[0, :NBLK]

    xs = _make_sc_scatter()(x, inv0, inv1)

    wsg_bf = w_sh_gate.astype(jnp.bfloat16)
    wsu_bf = w_sh_up.astype(jnp.bfloat16)
    wsd_bf = w_sh_down.astype(jnp.bfloat16)
    sh = pl.pallas_call(
        _shared_body,
        grid=(NT,),
        in_specs=[
            pl.BlockSpec((TBLK, D), lambda t: (t, 0)),
            pl.BlockSpec((D, FS), lambda t: (0, 0)),
            pl.BlockSpec((D, FS), lambda t: (0, 0)),
            pl.BlockSpec((FS, D), lambda t: (0, 0)),
        ],
        out_specs=pl.BlockSpec((TBLK, D), lambda t: (t, 0)),
        out_shape=jax.ShapeDtypeStruct((T, D), jnp.bfloat16),
    )(x, wsg_bf, wsu_bf, wsd_bf)

    wg_bf = w_gate_fp8.astype(jnp.bfloat16)
    wu_bf = w_up_fp8.astype(jnp.bfloat16)
    wd_bf = w_down_fp8.astype(jnp.bfloat16)

    ys = pl.pallas_call(
        _gemm_body,
        grid_spec=pltpu.PrefetchScalarGridSpec(
            num_scalar_prefetch=4,
            grid=(NBLK,),
            in_specs=[
                pl.BlockSpec((BLK, D), lambda b, beid, sg, su, sd: (b, 0)),
                pl.BlockSpec((1, D, F),
                             lambda b, beid, sg, su, sd: (beid[b], 0, 0)),
                pl.BlockSpec((1, D, F),
                             lambda b, beid, sg, su, sd: (beid[b], 0, 0)),
                pl.BlockSpec((1, F, D),
                             lambda b, beid, sg, su, sd: (beid[b], 0, 0)),
            ],
            out_specs=pl.BlockSpec((BLK, D),
                                   lambda b, beid, sg, su, sd: (b, 0)),
        ),
        out_shape=jax.ShapeDtypeStruct((NSLOT, D), jnp.float32),
    )(beid, s_g, s_u, s_d, xs, wg_bf, wu_bf, wd_bf)

    ytm = _make_sc_gather()(ys, inv0, inv1)

    out = pl.pallas_call(
        _comb_body,
        grid=(NT,),
        in_specs=[
            pl.BlockSpec((TBLK, D), lambda t: (t, 0)),
            pl.BlockSpec((TBLK, D), lambda t: (t, 0)),
            pl.BlockSpec((TBLK, D), lambda t: (t + NT, 0)),
            pl.BlockSpec((TBLK, 1), lambda t: (t, 0)),
            pl.BlockSpec((TBLK, 1), lambda t: (t, 0)),
        ],
        out_specs=pl.BlockSpec((TBLK, D), lambda t: (t, 0)),
        out_shape=jax.ShapeDtypeStruct((T, D), jnp.float32),
    )(sh, ytm, ytm, wt0_col, wt1_col)

    return out.reshape(shape)


# speedup vs baseline: 3.7457x; 1.0470x over previous
import functools

import jax
import jax.numpy as jnp
from jax import lax
from jax.experimental import pallas as pl
from jax.experimental.pallas import tpu as pltpu
from jax.experimental.pallas import tpu_sc as plsc

E = 8
TOP_K = 2
D = 1024
F = 512
FS = 1024
ROUTED_SCALE = 2.5

T = 2048
A = T * TOP_K
BLK = 256
NBLK = A // BLK + E
NSLOT = NBLK * BLK
TBLK = 256
NT = T // TBLK


def _inc_cumsum0(m):
    c = m
    d = 1
    while d < c.shape[0]:
        z = jnp.zeros((d, c.shape[1]), c.dtype)
        c = c + jnp.concatenate([z, c[:-d]], axis=0)
        d *= 2
    return c


def _col_to_2d(col, rows, lanes):
    n = col.shape[0]
    lane = lax.broadcasted_iota(jnp.int32, (n, lanes), 1)
    sub = lax.broadcasted_iota(jnp.int32, (n, lanes), 0)
    b = jnp.where(lane == sub % lanes, col, 0.0)
    return jnp.sum(b.reshape(rows, lanes, lanes), axis=1)


def _rows_to_col(w2):
    r = w2.shape[0]
    n = r * 128
    rep = jnp.broadcast_to(w2[:, None, :], (r, 128, 128)).reshape(n, 128)
    lane = lax.broadcasted_iota(jnp.int32, (n, 128), 1)
    sub = lax.broadcasted_iota(jnp.int32, (n, 128), 0)
    return jnp.sum(jnp.where(lane == sub % 128, rep, 0.0), axis=-1,
                   keepdims=True)


def _plan_body(x_ref, wr_ref, inv0_ref, inv1_ref, wt0_ref, wt1_ref,
               beid_ref):
    x = x_ref[...]
    logits = jnp.dot(x, wr_ref[...], preferred_element_type=jnp.float32)
    s = jax.nn.softmax(logits, axis=-1)
    lane8 = lax.broadcasted_iota(jnp.int32, s.shape, 1)
    m1 = jnp.max(s, axis=-1, keepdims=True)
    i1 = jnp.min(jnp.where(s == m1, lane8, E), axis=-1, keepdims=True)
    oh1 = (lane8 == i1).astype(jnp.float32)
    s2 = jnp.where(lane8 == i1, -jnp.inf, s)
    m2 = jnp.max(s2, axis=-1, keepdims=True)
    i2 = jnp.min(jnp.where(s2 == m2, lane8, E), axis=-1, keepdims=True)
    oh2 = (lane8 == i2).astype(jnp.float32)
    v1 = jnp.sum(s * oh1, axis=-1, keepdims=True)
    v2 = jnp.sum(s * oh2, axis=-1, keepdims=True)

    inc1 = _inc_cumsum0(oh1)
    inc2 = _inc_cumsum0(oh2)
    exc1 = inc1 - oh1
    exc2 = inc2 - oh2
    cnt1 = inc1[T - 1:T, :]
    cnt2 = inc2[T - 1:T, :]
    cnt = cnt1 + cnt2
    nb = jnp.ceil(cnt * (1.0 / BLK))
    ii = lax.broadcasted_iota(jnp.int32, (E, E), 0)
    jj = lax.broadcasted_iota(jnp.int32, (E, E), 1)
    tri = (ii <= jj).astype(jnp.float32)
    bcum = jnp.dot(nb, tri)
    boff = bcum - nb
    soff = boff * float(BLK)

    soff1 = jnp.sum(oh1 * soff, axis=-1, keepdims=True)
    soff2 = jnp.sum(oh2 * (soff + cnt1), axis=-1, keepdims=True)
    rank1 = jnp.sum(oh1 * exc1, axis=-1, keepdims=True)
    rank2 = jnp.sum(oh2 * exc2, axis=-1, keepdims=True)
    slot0 = soff1 + rank1
    slot1 = soff2 + rank2

    inv0_ref[...] = _col_to_2d(slot0, T // 128, 128).astype(jnp.int32)
    inv1_ref[...] = _col_to_2d(slot1, T // 128, 128).astype(jnp.int32)
    wt0_ref[...] = v1 * ROUTED_SCALE
    wt1_ref[...] = v2 * ROUTED_SCALE

    biota = lax.broadcasted_iota(jnp.int32, (1, 128), 1).astype(jnp.float32)
    acc = jnp.zeros((1, 128), jnp.float32)
    for e in range(E):
        bce = jnp.sum(jnp.where(lane8[:1] == e, bcum, 0.0))
        acc = acc + (biota >= bce).astype(jnp.float32)
    beid_ref[...] = jnp.minimum(acc, float(E - 1)).astype(jnp.int32)


def _gemm_body(beid_ref, sg_ref, su_ref, sd_ref, xs_ref, wg_ref, wu_ref,
               wd_ref, ys_ref):
    b = pl.program_id(0)
    e = beid_ref[b]
    xb = xs_ref[...].astype(jnp.bfloat16)
    g = jnp.dot(xb, wg_ref[0], preferred_element_type=jnp.float32) * sg_ref[e]
    u = jnp.dot(xb, wu_ref[0], preferred_element_type=jnp.float32) * su_ref[e]
    h = (jax.nn.silu(g) * u).astype(jnp.bfloat16)
    ys_ref[...] = (jnp.dot(h, wd_ref[0], preferred_element_type=jnp.float32)
                   * sd_ref[e])


def _comb_body(x_ref, wg_ref, wu_ref, wd_ref, y0_ref, y1_ref, wt0_ref,
               wt1_ref, out_ref):
    x = x_ref[...].astype(jnp.bfloat16)
    g = jnp.dot(x, wg_ref[...], preferred_element_type=jnp.float32)
    u = jnp.dot(x, wu_ref[...], preferred_element_type=jnp.float32)
    h = (jax.nn.silu(g) * u).astype(jnp.bfloat16)
    sh = jnp.dot(h, wd_ref[...], preferred_element_type=jnp.float32)
    out_ref[...] = sh + wt0_ref[...] * y0_ref[...] + wt1_ref[...] * y1_ref[...]


def _sc_info():
    info = plsc.get_sparse_core_info()
    return info.num_cores, info.num_subcores


def _make_sc_scatter():
    nc, ns = _sc_info()
    nw = nc * ns
    tpw = T // nw
    mesh = plsc.VectorSubcoreMesh(core_axis_name="c", subcore_axis_name="s")

    @functools.partial(
        pl.kernel, mesh=mesh,
        out_type=jax.ShapeDtypeStruct((NSLOT, D), jnp.float32),
        scratch_types=[
            pltpu.VMEM((tpw,), jnp.int32),
            pltpu.VMEM((tpw,), jnp.int32),
            pltpu.VMEM((tpw, D), jnp.float32),
            pltpu.SemaphoreType.DMA,
            pltpu.SemaphoreType.DMA,
        ],
    )
    def sc_scatter(x_hbm, inv0_hbm, inv1_hbm, xs_hbm, idx0_v, idx1_v,
                   rows_v, sem, sem2):
        wid = lax.axis_index("s") * nc + lax.axis_index("c")
        base = wid * tpw
        ld = pltpu.async_copy(x_hbm.at[pl.ds(base, tpw)], rows_v, sem2)
        pltpu.sync_copy(inv0_hbm.at[pl.ds(base, tpw)], idx0_v)
        pltpu.sync_copy(inv1_hbm.at[pl.ds(base, tpw)], idx1_v)
        ld.wait()
        c0 = pltpu.async_copy(rows_v, xs_hbm.at[idx0_v], sem)
        c1 = pltpu.async_copy(rows_v, xs_hbm.at[idx1_v], sem)
        c0.wait()
        c1.wait()

    return sc_scatter


def _make_sc_gather():
    nc, ns = _sc_info()
    nw = nc * ns
    apw = T // nw
    mesh = plsc.VectorSubcoreMesh(core_axis_name="c", subcore_axis_name="s")

    ch = apw // 2

    @functools.partial(
        pl.kernel, mesh=mesh,
        out_type=jax.ShapeDtypeStruct((A, D), jnp.float32),
        scratch_types=[
            pltpu.VMEM((ch,), jnp.int32),
            pltpu.VMEM((ch,), jnp.int32),
            pltpu.VMEM((ch,), jnp.int32),
            pltpu.VMEM((ch,), jnp.int32),
            pltpu.VMEM((ch, D), jnp.float32),
            pltpu.VMEM((ch, D), jnp.float32),
            pltpu.SemaphoreType.DMA,
            pltpu.SemaphoreType.DMA,
            pltpu.SemaphoreType.DMA,
        ],
    )
    def sc_gather(ys_hbm, inv0_hbm, inv1_hbm, ytm_hbm, ia_v, ib_v, ic_v,
                  id_v, rows0_v, rows1_v, sg, so0, so1):
        wid = lax.axis_index("s") * nc + lax.axis_index("c")
        base = wid * apw
        pltpu.sync_copy(inv0_hbm.at[pl.ds(base, ch)], ia_v)
        pltpu.sync_copy(inv0_hbm.at[pl.ds(base + ch, ch)], ib_v)
        pltpu.sync_copy(inv1_hbm.at[pl.ds(base, ch)], ic_v)
        pltpu.sync_copy(inv1_hbm.at[pl.ds(base + ch, ch)], id_v)
        bufs = (rows0_v, rows1_v)
        osems = (so0, so1)
        descs = ((ia_v, base), (ib_v, base + ch),
                 (ic_v, T + base), (id_v, T + base + ch))
        outs = [None, None]
        for i, (iv, dst) in enumerate(descs):
            b = i % 2
            if outs[b] is not None:
                outs[b].wait()
            pltpu.async_copy(ys_hbm.at[iv], bufs[b], sg).wait()
            outs[b] = pltpu.async_copy(
                bufs[b], ytm_hbm.at[pl.ds(dst, ch)], osems[b])
        outs[0].wait()
        outs[1].wait()

    return sc_gather


def kernel(hidden_states, w_router, w_gate_fp8, w_up_fp8, w_down_fp8,
           s_g, s_u, s_d, w_sh_gate, w_sh_up, w_sh_down):
    shape = hidden_states.shape
    x = hidden_states.reshape(-1, shape[-1])

    inv0_2d, inv1_2d, wt0_col, wt1_col, beid_2d = pl.pallas_call(
        _plan_body,
        in_specs=[
            pl.BlockSpec((T, D), lambda: (0, 0)),
            pl.BlockSpec((D, E), lambda: (0, 0)),
        ],
        out_specs=[
            pl.BlockSpec((T // 128, 128), lambda: (0, 0)),
            pl.BlockSpec((T // 128, 128), lambda: (0, 0)),
            pl.BlockSpec((T, 1), lambda: (0, 0)),
            pl.BlockSpec((T, 1), lambda: (0, 0)),
            pl.BlockSpec((1, 128), lambda: (0, 0)),
        ],
        out_shape=[
            jax.ShapeDtypeStruct((T // 128, 128), jnp.int32),
            jax.ShapeDtypeStruct((T // 128, 128), jnp.int32),
            jax.ShapeDtypeStruct((T, 1), jnp.float32),
            jax.ShapeDtypeStruct((T, 1), jnp.float32),
            jax.ShapeDtypeStruct((1, 128), jnp.int32),
        ],
    )(x, w_router)

    inv0 = inv0_2d.reshape(T)
    inv1 = inv1_2d.reshape(T)
    beid = beid_2d[0, :NBLK]

    xs = _make_sc_scatter()(x, inv0, inv1)

    wsg_bf = w_sh_gate.astype(jnp.bfloat16)
    wsu_bf = w_sh_up.astype(jnp.bfloat16)
    wsd_bf = w_sh_down.astype(jnp.bfloat16)

    wg_bf = w_gate_fp8.astype(jnp.bfloat16)
    wu_bf = w_up_fp8.astype(jnp.bfloat16)
    wd_bf = w_down_fp8.astype(jnp.bfloat16)

    ys = pl.pallas_call(
        _gemm_body,
        grid_spec=pltpu.PrefetchScalarGridSpec(
            num_scalar_prefetch=4,
            grid=(NBLK,),
            in_specs=[
                pl.BlockSpec((BLK, D), lambda b, beid, sg, su, sd: (b, 0)),
                pl.BlockSpec((1, D, F),
                             lambda b, beid, sg, su, sd: (beid[b], 0, 0)),
                pl.BlockSpec((1, D, F),
                             lambda b, beid, sg, su, sd: (beid[b], 0, 0)),
                pl.BlockSpec((1, F, D),
                             lambda b, beid, sg, su, sd: (beid[b], 0, 0)),
            ],
            out_specs=pl.BlockSpec((BLK, D),
                                   lambda b, beid, sg, su, sd: (b, 0)),
        ),
        out_shape=jax.ShapeDtypeStruct((NSLOT, D), jnp.float32),
    )(beid, s_g, s_u, s_d, xs, wg_bf, wu_bf, wd_bf)

    ytm = _make_sc_gather()(ys, inv0, inv1)

    out = pl.pallas_call(
        _comb_body,
        grid=(NT,),
        in_specs=[
            pl.BlockSpec((TBLK, D), lambda t: (t, 0)),
            pl.BlockSpec((D, FS), lambda t: (0, 0)),
            pl.BlockSpec((D, FS), lambda t: (0, 0)),
            pl.BlockSpec((FS, D), lambda t: (0, 0)),
            pl.BlockSpec((TBLK, D), lambda t: (t, 0)),
            pl.BlockSpec((TBLK, D), lambda t: (t + NT, 0)),
            pl.BlockSpec((TBLK, 1), lambda t: (t, 0)),
            pl.BlockSpec((TBLK, 1), lambda t: (t, 0)),
        ],
        out_specs=pl.BlockSpec((TBLK, D), lambda t: (t, 0)),
        out_shape=jax.ShapeDtypeStruct((T, D), jnp.float32),
    )(x, wsg_bf, wsu_bf, wsd_bf, ytm, ytm, wt0_col, wt1_col)

    return out.reshape(shape)


# speedup vs baseline: 4.0973x; 1.0939x over previous
import functools

import jax
import jax.numpy as jnp
from jax import lax
from jax.experimental import pallas as pl
from jax.experimental.pallas import tpu as pltpu
from jax.experimental.pallas import tpu_sc as plsc

E = 8
TOP_K = 2
D = 1024
F = 512
FS = 1024
ROUTED_SCALE = 2.5

T = 2048
A = T * TOP_K
BLK = 256
NBLK = A // BLK + E
NSLOT = NBLK * BLK
TBLK = 256
NT = T // TBLK


def _inc_cumsum0(m):
    c = m
    d = 1
    while d < c.shape[0]:
        z = jnp.zeros((d, c.shape[1]), c.dtype)
        c = c + jnp.concatenate([z, c[:-d]], axis=0)
        d *= 2
    return c


def _col_to_2d(col, rows, lanes):
    n = col.shape[0]
    lane = lax.broadcasted_iota(jnp.int32, (n, lanes), 1)
    sub = lax.broadcasted_iota(jnp.int32, (n, lanes), 0)
    b = jnp.where(lane == sub % lanes, col, 0.0)
    return jnp.sum(b.reshape(rows, lanes, lanes), axis=1)


def _rows_to_col(w2):
    r = w2.shape[0]
    n = r * 128
    rep = jnp.broadcast_to(w2[:, None, :], (r, 128, 128)).reshape(n, 128)
    lane = lax.broadcasted_iota(jnp.int32, (n, 128), 1)
    sub = lax.broadcasted_iota(jnp.int32, (n, 128), 0)
    return jnp.sum(jnp.where(lane == sub % 128, rep, 0.0), axis=-1,
                   keepdims=True)


def _plan_body(x_ref, wr_ref, inv0_ref, inv1_ref, wt0_ref, wt1_ref,
               beid_ref):
    x = x_ref[...]
    logits = jnp.dot(x, wr_ref[...], preferred_element_type=jnp.float32)
    s = jax.nn.softmax(logits, axis=-1)
    lane8 = lax.broadcasted_iota(jnp.int32, s.shape, 1)
    m1 = jnp.max(s, axis=-1, keepdims=True)
    i1 = jnp.min(jnp.where(s == m1, lane8, E), axis=-1, keepdims=True)
    oh1 = (lane8 == i1).astype(jnp.float32)
    s2 = jnp.where(lane8 == i1, -jnp.inf, s)
    m2 = jnp.max(s2, axis=-1, keepdims=True)
    i2 = jnp.min(jnp.where(s2 == m2, lane8, E), axis=-1, keepdims=True)
    oh2 = (lane8 == i2).astype(jnp.float32)
    v1 = jnp.sum(s * oh1, axis=-1, keepdims=True)
    v2 = jnp.sum(s * oh2, axis=-1, keepdims=True)

    inc1 = _inc_cumsum0(oh1)
    inc2 = _inc_cumsum0(oh2)
    exc1 = inc1 - oh1
    exc2 = inc2 - oh2
    cnt1 = inc1[T - 1:T, :]
    cnt2 = inc2[T - 1:T, :]
    cnt = cnt1 + cnt2
    nb = jnp.ceil(cnt * (1.0 / BLK))
    ii = lax.broadcasted_iota(jnp.int32, (E, E), 0)
    jj = lax.broadcasted_iota(jnp.int32, (E, E), 1)
    tri = (ii <= jj).astype(jnp.float32)
    bcum = jnp.dot(nb, tri)
    boff = bcum - nb
    soff = boff * float(BLK)

    soff1 = jnp.sum(oh1 * soff, axis=-1, keepdims=True)
    soff2 = jnp.sum(oh2 * (soff + cnt1), axis=-1, keepdims=True)
    rank1 = jnp.sum(oh1 * exc1, axis=-1, keepdims=True)
    rank2 = jnp.sum(oh2 * exc2, axis=-1, keepdims=True)
    slot0 = soff1 + rank1
    slot1 = soff2 + rank2

    inv0_ref[...] = _col_to_2d(slot0, T // 128, 128).astype(jnp.int32)
    inv1_ref[...] = _col_to_2d(slot1, T // 128, 128).astype(jnp.int32)
    wt0_ref[...] = v1 * ROUTED_SCALE
    wt1_ref[...] = v2 * ROUTED_SCALE

    biota = lax.broadcasted_iota(jnp.int32, (1, 128), 1).astype(jnp.float32)
    acc = jnp.zeros((1, 128), jnp.float32)
    for e in range(E):
        bce = jnp.sum(jnp.where(lane8[:1] == e, bcum, 0.0))
        acc = acc + (biota >= bce).astype(jnp.float32)
    beid_ref[...] = jnp.minimum(acc, float(E - 1)).astype(jnp.int32)


def _gemm_body(beid_ref, sg_ref, su_ref, sd_ref, xs_ref, wg_ref, wu_ref,
               wd_ref, ys_ref):
    b = pl.program_id(0)
    e = beid_ref[b]
    xb = xs_ref[...].astype(jnp.bfloat16)
    wg = wg_ref[0].astype(jnp.bfloat16)
    wu = wu_ref[0].astype(jnp.bfloat16)
    wd = wd_ref[0].astype(jnp.bfloat16)
    g = jnp.dot(xb, wg, preferred_element_type=jnp.float32) * sg_ref[e]
    u = jnp.dot(xb, wu, preferred_element_type=jnp.float32) * su_ref[e]
    h = (jax.nn.silu(g) * u).astype(jnp.bfloat16)
    ys_ref[...] = (jnp.dot(h, wd, preferred_element_type=jnp.float32)
                   * sd_ref[e])


def _comb_body(x_ref, wg_ref, wu_ref, wd_ref, y0_ref, y1_ref, wt0_ref,
               wt1_ref, out_ref):
    x = x_ref[...].astype(jnp.bfloat16)
    g = jnp.dot(x, wg_ref[...].astype(jnp.bfloat16),
                preferred_element_type=jnp.float32)
    u = jnp.dot(x, wu_ref[...].astype(jnp.bfloat16),
                preferred_element_type=jnp.float32)
    h = (jax.nn.silu(g) * u).astype(jnp.bfloat16)
    sh = jnp.dot(h, wd_ref[...].astype(jnp.bfloat16),
                 preferred_element_type=jnp.float32)
    out_ref[...] = sh + wt0_ref[...] * y0_ref[...] + wt1_ref[...] * y1_ref[...]


def _sc_info():
    info = plsc.get_sparse_core_info()
    return info.num_cores, info.num_subcores


def _make_sc_scatter():
    nc, ns = _sc_info()
    nw = nc * ns
    tpw = T // nw
    mesh = plsc.VectorSubcoreMesh(core_axis_name="c", subcore_axis_name="s")

    @functools.partial(
        pl.kernel, mesh=mesh,
        out_type=jax.ShapeDtypeStruct((NSLOT, D), jnp.float32),
        scratch_types=[
            pltpu.VMEM((tpw,), jnp.int32),
            pltpu.VMEM((tpw,), jnp.int32),
            pltpu.VMEM((tpw, D), jnp.float32),
            pltpu.SemaphoreType.DMA,
            pltpu.SemaphoreType.DMA,
        ],
    )
    def sc_scatter(x_hbm, inv0_hbm, inv1_hbm, xs_hbm, idx0_v, idx1_v,
                   rows_v, sem, sem2):
        wid = lax.axis_index("s") * nc + lax.axis_index("c")
        base = wid * tpw
        ld = pltpu.async_copy(x_hbm.at[pl.ds(base, tpw)], rows_v, sem2)
        pltpu.sync_copy(inv0_hbm.at[pl.ds(base, tpw)], idx0_v)
        pltpu.sync_copy(inv1_hbm.at[pl.ds(base, tpw)], idx1_v)
        ld.wait()
        c0 = pltpu.async_copy(rows_v, xs_hbm.at[idx0_v], sem)
        c1 = pltpu.async_copy(rows_v, xs_hbm.at[idx1_v], sem)
        c0.wait()
        c1.wait()

    return sc_scatter


def _make_sc_gather():
    nc, ns = _sc_info()
    nw = nc * ns
    apw = T // nw
    mesh = plsc.VectorSubcoreMesh(core_axis_name="c", subcore_axis_name="s")

    ch = apw // 2

    @functools.partial(
        pl.kernel, mesh=mesh,
        out_type=jax.ShapeDtypeStruct((A, D), jnp.float32),
        scratch_types=[
            pltpu.VMEM((ch,), jnp.int32),
            pltpu.VMEM((ch,), jnp.int32),
            pltpu.VMEM((ch,), jnp.int32),
            pltpu.VMEM((ch,), jnp.int32),
            pltpu.VMEM((ch, D), jnp.float32),
            pltpu.VMEM((ch, D), jnp.float32),
            pltpu.SemaphoreType.DMA,
            pltpu.SemaphoreType.DMA,
            pltpu.SemaphoreType.DMA,
        ],
    )
    def sc_gather(ys_hbm, inv0_hbm, inv1_hbm, ytm_hbm, ia_v, ib_v, ic_v,
                  id_v, rows0_v, rows1_v, sg, so0, so1):
        wid = lax.axis_index("s") * nc + lax.axis_index("c")
        base = wid * apw
        pltpu.sync_copy(inv0_hbm.at[pl.ds(base, ch)], ia_v)
        pltpu.sync_copy(inv0_hbm.at[pl.ds(base + ch, ch)], ib_v)
        pltpu.sync_copy(inv1_hbm.at[pl.ds(base, ch)], ic_v)
        pltpu.sync_copy(inv1_hbm.at[pl.ds(base + ch, ch)], id_v)
        bufs = (rows0_v, rows1_v)
        osems = (so0, so1)
        descs = ((ia_v, base), (ib_v, base + ch),
                 (ic_v, T + base), (id_v, T + base + ch))
        outs = [None, None]
        for i, (iv, dst) in enumerate(descs):
            b = i % 2
            if outs[b] is not None:
                outs[b].wait()
            pltpu.async_copy(ys_hbm.at[iv], bufs[b], sg).wait()
            outs[b] = pltpu.async_copy(
                bufs[b], ytm_hbm.at[pl.ds(dst, ch)], osems[b])
        outs[0].wait()
        outs[1].wait()

    return sc_gather


def kernel(hidden_states, w_router, w_gate_fp8, w_up_fp8, w_down_fp8,
           s_g, s_u, s_d, w_sh_gate, w_sh_up, w_sh_down):
    shape = hidden_states.shape
    x = hidden_states.reshape(-1, shape[-1])

    inv0_2d, inv1_2d, wt0_col, wt1_col, beid_2d = pl.pallas_call(
        _plan_body,
        in_specs=[
            pl.BlockSpec((T, D), lambda: (0, 0)),
            pl.BlockSpec((D, E), lambda: (0, 0)),
        ],
        out_specs=[
            pl.BlockSpec((T // 128, 128), lambda: (0, 0)),
            pl.BlockSpec((T // 128, 128), lambda: (0, 0)),
            pl.BlockSpec((T, 1), lambda: (0, 0)),
            pl.BlockSpec((T, 1), lambda: (0, 0)),
            pl.BlockSpec((1, 128), lambda: (0, 0)),
        ],
        out_shape=[
            jax.ShapeDtypeStruct((T // 128, 128), jnp.int32),
            jax.ShapeDtypeStruct((T // 128, 128), jnp.int32),
            jax.ShapeDtypeStruct((T, 1), jnp.float32),
            jax.ShapeDtypeStruct((T, 1), jnp.float32),
            jax.ShapeDtypeStruct((1, 128), jnp.int32),
        ],
    )(x, w_router)

    inv0 = inv0_2d.reshape(T)
    inv1 = inv1_2d.reshape(T)
    beid = beid_2d[0, :NBLK]

    xs = _make_sc_scatter()(x, inv0, inv1)

    ys = pl.pallas_call(
        _gemm_body,
        grid_spec=pltpu.PrefetchScalarGridSpec(
            num_scalar_prefetch=4,
            grid=(NBLK,),
            in_specs=[
                pl.BlockSpec((BLK, D), lambda b, beid, sg, su, sd: (b, 0)),
                pl.BlockSpec((1, D, F),
                             lambda b, beid, sg, su, sd: (beid[b], 0, 0)),
                pl.BlockSpec((1, D, F),
                             lambda b, beid, sg, su, sd: (beid[b], 0, 0)),
                pl.BlockSpec((1, F, D),
                             lambda b, beid, sg, su, sd: (beid[b], 0, 0)),
            ],
            out_specs=pl.BlockSpec((BLK, D),
                                   lambda b, beid, sg, su, sd: (b, 0)),
        ),
        out_shape=jax.ShapeDtypeStruct((NSLOT, D), jnp.float32),
    )(beid, s_g, s_u, s_d, xs, w_gate_fp8, w_up_fp8, w_down_fp8)

    ytm = _make_sc_gather()(ys, inv0, inv1)

    out = pl.pallas_call(
        _comb_body,
        grid=(NT,),
        in_specs=[
            pl.BlockSpec((TBLK, D), lambda t: (t, 0)),
            pl.BlockSpec((D, FS), lambda t: (0, 0)),
            pl.BlockSpec((D, FS), lambda t: (0, 0)),
            pl.BlockSpec((FS, D), lambda t: (0, 0)),
            pl.BlockSpec((TBLK, D), lambda t: (t, 0)),
            pl.BlockSpec((TBLK, D), lambda t: (t + NT, 0)),
            pl.BlockSpec((TBLK, 1), lambda t: (t, 0)),
            pl.BlockSpec((TBLK, 1), lambda t: (t, 0)),
        ],
        out_specs=pl.BlockSpec((TBLK, D), lambda t: (t, 0)),
        out_shape=jax.ShapeDtypeStruct((T, D), jnp.float32),
    )(x, w_sh_gate, w_sh_up, w_sh_down, ytm, ytm, wt0_col, wt1_col)

    return out.reshape(shape)
